# baseline (speedup 1.0000x reference)
import functools

import jax
import jax.numpy as jnp
from jax import lax
from jax.experimental import pallas as pl
from jax.experimental.pallas import tpu as pltpu
from jax.experimental.pallas import tpu_sc as plsc

NC, NS, L = 2, 16, 16
NW = NC * NS
BATCH, HIST, D = 16384, 50, 128
B = BATCH * HIST
NHALF = 2
BATCH_H = BATCH // NHALF
BPWB = BATCH_H // NW
CHB = 4
LOOK = CHB * HIST
NCHUNK = BPWB // CHB
NBUF = 4
ROUNDS = NCHUNK // NBUF


def _body(xs_hbm, ys_hbm, zs_hbm, table_hbm, out_hbm,
          cx_v, cy_v, cz_v, idx_v, rows_v, *sems):
    csem = sems[0:NBUF]
    gsem = sems[NBUF:2 * NBUF]
    osem = sems[2 * NBUF:3 * NBUF]
    wid = lax.axis_index("s") * NC + lax.axis_index("c")
    batch0 = wid * BPWB
    i0 = wid * BPWB * HIST

    def fire_coords(g, b):
        s = pl.ds(i0 + g * LOOK, LOOK)
        d = pl.ds(b * LOOK, LOOK)
        pltpu.async_copy(xs_hbm.at[s], cx_v.at[d], csem[b])
        pltpu.async_copy(ys_hbm.at[s], cy_v.at[d], csem[b])
        pltpu.async_copy(zs_hbm.at[s], cz_v.at[d], csem[b])

    def wait_coords(b):
        d = pl.ds(b * LOOK, LOOK)
        for ref in (cx_v, cy_v, cz_v):
            pltpu.make_async_copy(xs_hbm.at[pl.ds(0, LOOK)], ref.at[d],
                                  csem[b]).wait()

    _OFFS = [16 * j for j in range(12)] + [LOOK - L]

    def compute_idx(b):
        for o in _OFFS:
            s = pl.ds(b * LOOK + o, L)
            idx_v[s] = (cx_v[s] * 16 + cy_v[s]) * 16 + cz_v[s]

    def fire_gather(b):
        pltpu.async_copy(table_hbm.at[idx_v.at[pl.ds(b * LOOK, 128)]],
                         rows_v.at[b, pl.ds(0, 128), :], gsem[b])
        pltpu.async_copy(table_hbm.at[idx_v.at[pl.ds(b * LOOK + 128, LOOK - 128)]],
                         rows_v.at[b, pl.ds(128, LOOK - 128), :], gsem[b])

    def wait_gather(b):
        pltpu.make_async_copy(table_hbm.at[idx_v.at[pl.ds(b * LOOK, 128)]],
                              rows_v.at[b, pl.ds(0, 128), :], gsem[b]).wait()
        pltpu.make_async_copy(table_hbm.at[idx_v.at[pl.ds(b * LOOK + 128, LOOK - 128)]],
                              rows_v.at[b, pl.ds(128, LOOK - 128), :],
                              gsem[b]).wait()

    def fire_out(g, b):
        for m in range(CHB):
            pltpu.async_copy(rows_v.at[b, pl.ds(m * HIST, HIST), :],
                             out_hbm.at[batch0 + g * CHB + m], osem[b])

    def wait_out(b):
        for m in range(CHB):
            pltpu.make_async_copy(rows_v.at[b, pl.ds(m * HIST, HIST), :],
                                  out_hbm.at[batch0], osem[b]).wait()

    for b in range(NBUF):
        fire_coords(b, b)

    def round_body(r, carry):
        for b in range(NBUF):
            g = r * NBUF + b
            wait_coords(b)
            compute_idx(b)

            @pl.when(r > 0)
            def _():
                wait_out(b)

            fire_gather(b)
            pb = (b - 1) % NBUF
            if b > 0:
                wait_gather(pb)
                fire_out(g - 1, pb)
            else:
                @pl.when(r > 0)
                def _():
                    wait_gather(pb)
                    fire_out(g - 1, pb)

            @pl.when(r < ROUNDS - 1)
            def _():
                fire_coords(g + NBUF, b)
        return carry

    lax.fori_loop(0, ROUNDS, round_body, 0)

    bl = (NCHUNK - 1) % NBUF
    wait_gather(bl)
    for m in range(CHB):
        pltpu.sync_copy(rows_v.at[bl, pl.ds(m * HIST, HIST), :],
                        out_hbm.at[batch0 + (NCHUNK - 1) * CHB + m])
    for b in range(NBUF):
        if b != bl:
            wait_out(b)


_gather = functools.partial(
    pl.kernel,
    out_type=jax.ShapeDtypeStruct((BATCH_H, HIST, D), jnp.float32),
    mesh=plsc.VectorSubcoreMesh(core_axis_name="c", subcore_axis_name="s"),
    scratch_types=(
        [
            pltpu.VMEM((NBUF * LOOK,), jnp.int32),
            pltpu.VMEM((NBUF * LOOK,), jnp.int32),
            pltpu.VMEM((NBUF * LOOK,), jnp.int32),
            pltpu.VMEM((NBUF * LOOK,), jnp.int32),
            pltpu.VMEM((NBUF, LOOK, D), jnp.float32),
        ]
        + [pltpu.SemaphoreType.DMA] * (3 * NBUF)
    ),
)(_body)


@jax.jit
def kernel(input, table):
    flat = input.reshape(B, 3)
    xs = flat[:, 0].reshape(B)
    ys = flat[:, 1].reshape(B)
    zs = flat[:, 2].reshape(B)
    bh = B // NHALF
    halves = [
        _gather(xs[h * bh:(h + 1) * bh], ys[h * bh:(h + 1) * bh],
                zs[h * bh:(h + 1) * bh], table)
        for h in range(NHALF)
    ]
    return jnp.concatenate(halves, axis=0)


# speedup vs baseline: 1.5573x; 1.5573x over previous
import functools

import jax
import jax.numpy as jnp
from jax import lax
from jax.experimental import pallas as pl
from jax.experimental.pallas import tpu as pltpu
from jax.experimental.pallas import tpu_sc as plsc

NC, NS, L = 2, 16, 16
NW = NC * NS
BATCH, HIST, D = 16384, 50, 128
B = BATCH * HIST
BPWB = BATCH // NW
CHB = 4
LOOK = CHB * HIST
NCHUNK = BPWB // CHB
NBUF = 4
ROUNDS = NCHUNK // NBUF


def _body(xs_hbm, ys_hbm, zs_hbm, table_hbm, out_hbm,
          cx_v, cy_v, cz_v, idx_v, rows_v, *sems):
    csem = sems[0:NBUF]
    gsem = sems[NBUF:2 * NBUF]
    osem = sems[2 * NBUF:3 * NBUF]
    wid = lax.axis_index("s") * NC + lax.axis_index("c")
    batch0 = wid * BPWB
    i0 = wid * BPWB * HIST

    def fire_coords(g, b):
        s = pl.ds(i0 + g * LOOK, LOOK)
        d = pl.ds(b * LOOK, LOOK)
        pltpu.async_copy(xs_hbm.at[s], cx_v.at[d], csem[b])
        pltpu.async_copy(ys_hbm.at[s], cy_v.at[d], csem[b])
        pltpu.async_copy(zs_hbm.at[s], cz_v.at[d], csem[b])

    def wait_coords(b):
        d = pl.ds(b * LOOK, LOOK)
        for ref in (cx_v, cy_v, cz_v):
            pltpu.make_async_copy(xs_hbm.at[pl.ds(0, LOOK)], ref.at[d],
                                  csem[b]).wait()

    _OFFS = [16 * j for j in range(12)] + [LOOK - L]

    def compute_idx(b):
        for o in _OFFS:
            s = pl.ds(b * LOOK + o, L)
            idx_v[s] = (cx_v[s] * 16 + cy_v[s]) * 16 + cz_v[s]

    def fire_gather(b):
        pltpu.async_copy(table_hbm.at[idx_v.at[pl.ds(b * LOOK, 128)]],
                         rows_v.at[b, pl.ds(0, 128), :], gsem[b])
        pltpu.async_copy(table_hbm.at[idx_v.at[pl.ds(b * LOOK + 128, LOOK - 128)]],
                         rows_v.at[b, pl.ds(128, LOOK - 128), :], gsem[b])

    def wait_gather(b):
        pltpu.make_async_copy(table_hbm.at[idx_v.at[pl.ds(b * LOOK, 128)]],
                              rows_v.at[b, pl.ds(0, 128), :], gsem[b]).wait()
        pltpu.make_async_copy(table_hbm.at[idx_v.at[pl.ds(b * LOOK + 128, LOOK - 128)]],
                              rows_v.at[b, pl.ds(128, LOOK - 128), :],
                              gsem[b]).wait()

    def fire_out(g, b):
        for m in range(CHB):
            pltpu.async_copy(rows_v.at[b, pl.ds(m * HIST, HIST), :],
                             out_hbm.at[batch0 + g * CHB + m], osem[b])

    def wait_out(b):
        for m in range(CHB):
            pltpu.make_async_copy(rows_v.at[b, pl.ds(m * HIST, HIST), :],
                                  out_hbm.at[batch0], osem[b]).wait()

    for b in range(NBUF):
        fire_coords(b, b)

    def round_body(r, carry):
        for b in range(NBUF):
            g = r * NBUF + b
            wait_coords(b)
            compute_idx(b)

            @pl.when(r > 0)
            def _():
                wait_out(b)

            fire_gather(b)
            pb = (b - 1) % NBUF
            if b > 0:
                wait_gather(pb)
                fire_out(g - 1, pb)
            else:
                @pl.when(r > 0)
                def _():
                    wait_gather(pb)
                    fire_out(g - 1, pb)

            @pl.when(r < ROUNDS - 1)
            def _():
                fire_coords(g + NBUF, b)
        return carry

    lax.fori_loop(0, ROUNDS, round_body, 0)

    bl = (NCHUNK - 1) % NBUF
    wait_gather(bl)
    for m in range(CHB):
        pltpu.sync_copy(rows_v.at[bl, pl.ds(m * HIST, HIST), :],
                        out_hbm.at[batch0 + (NCHUNK - 1) * CHB + m])
    for b in range(NBUF):
        if b != bl:
            wait_out(b)


_gather = functools.partial(
    pl.kernel,
    out_type=jax.ShapeDtypeStruct((BATCH, HIST, D), jnp.float32),
    mesh=plsc.VectorSubcoreMesh(core_axis_name="c", subcore_axis_name="s"),
    scratch_types=(
        [
            pltpu.VMEM((NBUF * LOOK,), jnp.int32),
            pltpu.VMEM((NBUF * LOOK,), jnp.int32),
            pltpu.VMEM((NBUF * LOOK,), jnp.int32),
            pltpu.VMEM((NBUF * LOOK,), jnp.int32),
            pltpu.VMEM((NBUF, LOOK, D), jnp.float32),
        ]
        + [pltpu.SemaphoreType.DMA] * (3 * NBUF)
    ),
)(_body)


@jax.jit
def kernel(input, table):
    flat = input.reshape(B, 3)
    xs = flat[:, 0].reshape(B)
    ys = flat[:, 1].reshape(B)
    zs = flat[:, 2].reshape(B)
    return _gather(xs, ys, zs, table)


# speedup vs baseline: 1.5596x; 1.0015x over previous
import functools

import jax
import jax.numpy as jnp
from jax import lax
from jax.experimental import pallas as pl
from jax.experimental.pallas import tpu as pltpu
from jax.experimental.pallas import tpu_sc as plsc

NC, NS, L = 2, 16, 16
NW = NC * NS
BATCH, HIST, D = 16384, 50, 128
B = BATCH * HIST
BPWB = BATCH // NW
CHB = 8
LOOK = CHB * HIST
NCHUNK = BPWB // CHB
NBUF = 2
ROUNDS = NCHUNK // NBUF


def _body(xs_hbm, ys_hbm, zs_hbm, table_hbm, out_hbm,
          cx_v, cy_v, cz_v, idx_v, rows_v, *sems):
    csem = sems[0:NBUF]
    gsem = sems[NBUF:2 * NBUF]
    osem = sems[2 * NBUF:3 * NBUF]
    wid = lax.axis_index("s") * NC + lax.axis_index("c")
    batch0 = wid * BPWB
    i0 = wid * BPWB * HIST

    def fire_coords(g, b):
        s = pl.ds(i0 + g * LOOK, LOOK)
        d = pl.ds(b * LOOK, LOOK)
        pltpu.async_copy(xs_hbm.at[s], cx_v.at[d], csem[b])
        pltpu.async_copy(ys_hbm.at[s], cy_v.at[d], csem[b])
        pltpu.async_copy(zs_hbm.at[s], cz_v.at[d], csem[b])

    def wait_coords(b):
        d = pl.ds(b * LOOK, LOOK)
        for ref in (cx_v, cy_v, cz_v):
            pltpu.make_async_copy(xs_hbm.at[pl.ds(0, LOOK)], ref.at[d],
                                  csem[b]).wait()

    _OFFS = [16 * j for j in range(LOOK // L)]

    def compute_idx(b):
        for o in _OFFS:
            s = pl.ds(b * LOOK + o, L)
            idx_v[s] = (cx_v[s] * 16 + cy_v[s]) * 16 + cz_v[s]

    _GS = [(k * 128, min(128, LOOK - k * 128)) for k in range((LOOK + 127) // 128)]

    def fire_gather(b):
        for o, n in _GS:
            pltpu.async_copy(table_hbm.at[idx_v.at[pl.ds(b * LOOK + o, n)]],
                             rows_v.at[b, pl.ds(o, n), :], gsem[b])

    def wait_gather(b):
        for o, n in _GS:
            pltpu.make_async_copy(table_hbm.at[idx_v.at[pl.ds(b * LOOK + o, n)]],
                                  rows_v.at[b, pl.ds(o, n), :], gsem[b]).wait()

    def fire_out(g, b):
        for m in range(CHB):
            pltpu.async_copy(rows_v.at[b, pl.ds(m * HIST, HIST), :],
                             out_hbm.at[batch0 + g * CHB + m], osem[b])

    def wait_out(b):
        for m in range(CHB):
            pltpu.make_async_copy(rows_v.at[b, pl.ds(m * HIST, HIST), :],
                                  out_hbm.at[batch0], osem[b]).wait()

    for b in range(NBUF):
        fire_coords(b, b)

    def round_body(r, carry):
        for b in range(NBUF):
            g = r * NBUF + b
            wait_coords(b)
            compute_idx(b)

            @pl.when(r > 0)
            def _():
                wait_out(b)

            fire_gather(b)
            pb = (b - 1) % NBUF
            if b > 0:
                wait_gather(pb)
                fire_out(g - 1, pb)
            else:
                @pl.when(r > 0)
                def _():
                    wait_gather(pb)
                    fire_out(g - 1, pb)

            @pl.when(r < ROUNDS - 1)
            def _():
                fire_coords(g + NBUF, b)
        return carry

    lax.fori_loop(0, ROUNDS, round_body, 0)

    bl = (NCHUNK - 1) % NBUF
    wait_gather(bl)
    for m in range(CHB):
        pltpu.sync_copy(rows_v.at[bl, pl.ds(m * HIST, HIST), :],
                        out_hbm.at[batch0 + (NCHUNK - 1) * CHB + m])
    for b in range(NBUF):
        if b != bl:
            wait_out(b)


_gather = functools.partial(
    pl.kernel,
    out_type=jax.ShapeDtypeStruct((BATCH, HIST, D), jnp.float32),
    mesh=plsc.VectorSubcoreMesh(core_axis_name="c", subcore_axis_name="s"),
    scratch_types=(
        [
            pltpu.VMEM((NBUF * LOOK,), jnp.int32),
            pltpu.VMEM((NBUF * LOOK,), jnp.int32),
            pltpu.VMEM((NBUF * LOOK,), jnp.int32),
            pltpu.VMEM((NBUF * LOOK,), jnp.int32),
            pltpu.VMEM((NBUF, LOOK, D), jnp.float32),
        ]
        + [pltpu.SemaphoreType.DMA] * (3 * NBUF)
    ),
)(_body)


@jax.jit
def kernel(input, table):
    flat = input.reshape(B, 3)
    xs = flat[:, 0].reshape(B)
    ys = flat[:, 1].reshape(B)
    zs = flat[:, 2].reshape(B)
    return _gather(xs, ys, zs, table)
